# baseline (speedup 1.0000x reference)
import jax
import jax.numpy as jnp
from jax.experimental import pallas as pl
from jax.experimental.pallas import tpu as pltpu

_H = 1024
_I = 4096
_E = 8
_K = 2
_BT = 128
_IC = 2048
_NI = _I // _IC


def _router_kernel(x_ref, wg_ref, idx_ref, w_ref):
    logits = jnp.dot(x_ref[...], wg_ref[...], preferred_element_type=jnp.float32)
    lane = jax.lax.broadcasted_iota(jnp.int32, logits.shape, 1)
    neg = jnp.float32(-1e30)
    lg = jnp.where(lane < _E, logits, neg)
    m1 = jnp.max(lg, axis=1, keepdims=True)
    i1 = jnp.min(jnp.where(lg == m1, lane, 128), axis=1, keepdims=True)
    lg2 = jnp.where(lane == i1, neg, lg)
    m2 = jnp.max(lg2, axis=1, keepdims=True)
    i2 = jnp.min(jnp.where(lg2 == m2, lane, 128), axis=1, keepdims=True)
    p1 = 1.0 / (1.0 + jnp.exp(m2 - m1))
    p2 = 1.0 - p1
    idx_ref[...] = jnp.where(lane == 0, i1, jnp.where(lane == 1, i2, 0))
    w_ref[...] = jnp.where(lane == 0, p1, jnp.where(lane == 1, p2, 0.0))


def _route(x_flat, Wg, T):
    wg_pad = jnp.zeros((_H, 128), jnp.float32).at[:, :_E].set(Wg)
    idx, w = pl.pallas_call(
        _router_kernel,
        grid=(T // _BT,),
        in_specs=[
            pl.BlockSpec((_BT, _H), lambda b: (b, 0)),
            pl.BlockSpec((_H, 128), lambda b: (0, 0)),
        ],
        out_specs=[
            pl.BlockSpec((_BT, 128), lambda b: (b, 0)),
            pl.BlockSpec((_BT, 128), lambda b: (b, 0)),
        ],
        out_shape=[
            jax.ShapeDtypeStruct((T, 128), jnp.int32),
            jax.ShapeDtypeStruct((T, 128), jnp.float32),
        ],
    )(x_flat, wg_pad)
    return idx[:, :_K], w[:, :_K]


def _ffn_kernel(be_ref, xs_ref, w1_ref, w3_ref, w2_ref, o_ref):
    del be_ref
    xb = xs_ref[...].astype(jnp.bfloat16)
    g = jnp.dot(xb, w1_ref[0].astype(jnp.bfloat16),
                preferred_element_type=jnp.float32)
    u = jnp.dot(xb, w3_ref[0].astype(jnp.bfloat16),
                preferred_element_type=jnp.float32)
    h = (g * jax.lax.logistic(g) * u).astype(jnp.bfloat16)
    o_ref[0] = jnp.dot(h, w2_ref[0].astype(jnp.bfloat16),
                       preferred_element_type=jnp.float32)


def _grouped_ffn(block_expert, xs, W1, W3, W2, NS, NB):
    grid_spec = pltpu.PrefetchScalarGridSpec(
        num_scalar_prefetch=1,
        grid=(_NI, NB),
        in_specs=[
            pl.BlockSpec((_BT, _H), lambda i, b, be: (b, 0)),
            pl.BlockSpec((1, _H, _IC), lambda i, b, be: (be[b], 0, i)),
            pl.BlockSpec((1, _H, _IC), lambda i, b, be: (be[b], 0, i)),
            pl.BlockSpec((1, _IC, _H), lambda i, b, be: (be[b], i, 0)),
        ],
        out_specs=pl.BlockSpec((1, _BT, _H), lambda i, b, be: (i, b, 0)),
    )
    return pl.pallas_call(
        _ffn_kernel,
        grid_spec=grid_spec,
        out_shape=jax.ShapeDtypeStruct((_NI, NS, _H), jnp.float32),
    )(block_expert, xs, W1, W3, W2)


def kernel(x, Wg, W1, W3, W2):
    B, S, H = x.shape
    T = B * S
    NA = T * _K
    NS = ((NA + _E * (_BT - 1) + _BT - 1) // _BT) * _BT
    NB = NS // _BT
    x_flat = x.reshape(T, H)

    idx, w = _route(x_flat, Wg, T)

    e_flat = idx.reshape(-1)
    order = jnp.argsort(e_flat, stable=True).astype(jnp.int32)
    sorted_e = e_flat[order]
    counts = jnp.bincount(e_flat, length=_E)
    padded = ((counts + _BT - 1) // _BT) * _BT
    pad_cum = jnp.cumsum(padded)
    expert_start = (pad_cum - padded).astype(jnp.int32)
    count_start = (jnp.cumsum(counts) - counts).astype(jnp.int32)
    j = jnp.arange(NA, dtype=jnp.int32)
    slot_of_sorted = expert_start[sorted_e] + (j - count_start[sorted_e])
    slot_token = jnp.zeros(NS, jnp.int32).at[slot_of_sorted].set(order // _K)
    pair_slot = jnp.zeros(NA, jnp.int32).at[order].set(slot_of_sorted)
    block_expert = jnp.minimum(
        jnp.searchsorted(pad_cum, jnp.arange(NB, dtype=jnp.int32) * _BT,
                         side="right"),
        _E - 1,
    ).astype(jnp.int32)

    xs = jnp.take(x_flat, slot_token, axis=0)
    ys = _grouped_ffn(block_expert, xs, W1, W3, W2, NS, NB)

    ysf = ys.reshape(_NI * NS, H)
    s = pair_slot.reshape(T, _K)
    idx4 = jnp.concatenate([s[:, 0], s[:, 1], s[:, 0] + NS, s[:, 1] + NS])
    g4 = jnp.take(ysf, idx4, axis=0).reshape(_NI * _K, T, H)
    out = (w[:, :1] * (g4[0] + g4[2]) + w[:, 1:] * (g4[1] + g4[3]))
    return out.reshape(B, S, H)


# speedup vs baseline: 1.0669x; 1.0669x over previous
import jax
import jax.numpy as jnp
from jax.experimental import pallas as pl
from jax.experimental.pallas import tpu as pltpu

_H = 1024
_I = 4096
_E = 8
_K = 2
_BT = 256
_IC = 2048
_NI = _I // _IC


def _router_kernel(x_ref, wg_ref, idx_ref, w_ref):
    logits = jnp.dot(x_ref[...], wg_ref[...], preferred_element_type=jnp.float32)
    lane = jax.lax.broadcasted_iota(jnp.int32, logits.shape, 1)
    neg = jnp.float32(-1e30)
    lg = jnp.where(lane < _E, logits, neg)
    m1 = jnp.max(lg, axis=1, keepdims=True)
    i1 = jnp.min(jnp.where(lg == m1, lane, 128), axis=1, keepdims=True)
    lg2 = jnp.where(lane == i1, neg, lg)
    m2 = jnp.max(lg2, axis=1, keepdims=True)
    i2 = jnp.min(jnp.where(lg2 == m2, lane, 128), axis=1, keepdims=True)
    p1 = 1.0 / (1.0 + jnp.exp(m2 - m1))
    p2 = 1.0 - p1
    idx_ref[...] = jnp.where(lane == 0, i1, jnp.where(lane == 1, i2, 0))
    w_ref[...] = jnp.where(lane == 0, p1, jnp.where(lane == 1, p2, 0.0))


def _route(x_flat, Wg, T):
    wg_pad = jnp.zeros((_H, 128), jnp.float32).at[:, :_E].set(Wg)
    idx, w = pl.pallas_call(
        _router_kernel,
        grid=(T // _BT,),
        in_specs=[
            pl.BlockSpec((_BT, _H), lambda b: (b, 0)),
            pl.BlockSpec((_H, 128), lambda b: (0, 0)),
        ],
        out_specs=[
            pl.BlockSpec((_BT, 128), lambda b: (b, 0)),
            pl.BlockSpec((_BT, 128), lambda b: (b, 0)),
        ],
        out_shape=[
            jax.ShapeDtypeStruct((T, 128), jnp.int32),
            jax.ShapeDtypeStruct((T, 128), jnp.float32),
        ],
    )(x_flat, wg_pad)
    return idx[:, :_K], w[:, :_K]


def _ffn_kernel(be_ref, xs_ref, w1_ref, w3_ref, w2_ref, o_ref):
    del be_ref
    xb = xs_ref[...].astype(jnp.bfloat16)
    g = jnp.dot(xb, w1_ref[0].astype(jnp.bfloat16),
                preferred_element_type=jnp.float32)
    u = jnp.dot(xb, w3_ref[0].astype(jnp.bfloat16),
                preferred_element_type=jnp.float32)
    h = (g * jax.lax.logistic(g) * u).astype(jnp.bfloat16)
    o_ref[0] = jnp.dot(h, w2_ref[0].astype(jnp.bfloat16),
                       preferred_element_type=jnp.float32)


def _grouped_ffn(block_expert, xs, W1, W3, W2, NS, NB):
    grid_spec = pltpu.PrefetchScalarGridSpec(
        num_scalar_prefetch=1,
        grid=(_NI, NB),
        in_specs=[
            pl.BlockSpec((_BT, _H), lambda i, b, be: (b, 0)),
            pl.BlockSpec((1, _H, _IC), lambda i, b, be: (be[b], 0, i)),
            pl.BlockSpec((1, _H, _IC), lambda i, b, be: (be[b], 0, i)),
            pl.BlockSpec((1, _IC, _H), lambda i, b, be: (be[b], i, 0)),
        ],
        out_specs=pl.BlockSpec((1, _BT, _H), lambda i, b, be: (i, b, 0)),
    )
    return pl.pallas_call(
        _ffn_kernel,
        grid_spec=grid_spec,
        out_shape=jax.ShapeDtypeStruct((_NI, NS, _H), jnp.float32),
    )(block_expert, xs, W1, W3, W2)


def kernel(x, Wg, W1, W3, W2):
    B, S, H = x.shape
    T = B * S
    NA = T * _K
    NS = ((NA + _E * (_BT - 1) + _BT - 1) // _BT) * _BT
    NB = NS // _BT
    x_flat = x.reshape(T, H)

    idx, w = _route(x_flat, Wg, T)

    e_flat = idx.reshape(-1)
    order = jnp.argsort(e_flat, stable=True).astype(jnp.int32)
    sorted_e = e_flat[order]
    counts = jnp.bincount(e_flat, length=_E)
    padded = ((counts + _BT - 1) // _BT) * _BT
    pad_cum = jnp.cumsum(padded)
    expert_start = (pad_cum - padded).astype(jnp.int32)
    count_start = (jnp.cumsum(counts) - counts).astype(jnp.int32)
    j = jnp.arange(NA, dtype=jnp.int32)
    slot_of_sorted = expert_start[sorted_e] + (j - count_start[sorted_e])
    slot_token = jnp.zeros(NS, jnp.int32).at[slot_of_sorted].set(order // _K)
    pair_slot = jnp.zeros(NA, jnp.int32).at[order].set(slot_of_sorted)
    block_expert = jnp.minimum(
        jnp.searchsorted(pad_cum, jnp.arange(NB, dtype=jnp.int32) * _BT,
                         side="right"),
        _E - 1,
    ).astype(jnp.int32)

    xs = jnp.take(x_flat, slot_token, axis=0)
    ys = jnp.sum(_grouped_ffn(block_expert, xs, W1, W3, W2, NS, NB), axis=0)

    s = pair_slot.reshape(T, _K)
    out = (w[:, :1] * jnp.take(ys, s[:, 0], axis=0)
           + w[:, 1:] * jnp.take(ys, s[:, 1], axis=0))
    return out.reshape(B, S, H)
